# baseline (device time: 54621 ns/iter reference)
import jax
import jax.numpy as jnp
from jax import lax
from jax.experimental import pallas as pl
from jax.experimental.pallas import tpu as pltpu

B, S, HD_LOCAL, F = 2, 1024, 1024, 2048
S_HALF = S // 2
Q = S_HALF // 4
NSUB = 8
QS = Q // NSUB


def kernel(O, Wo):
    O_flat = O.reshape(B, S, HD_LOCAL)

    def body(o_ref, wo_ref, out_ref, o_vmem, psend, precv, agsend, agrecv,
             o_copy_sems, y_send_sems, y_recv_sems, ag_send_sems,
             ag_recv_sems):
        my_x = lax.axis_index("x")
        my_y = lax.axis_index("y")
        my_z = lax.axis_index("z")
        y_peer = (my_x, 1 - my_y, my_z)
        x_nbr = (1 - my_x, my_y, my_z)
        z_nbr = (my_x, my_y, 1 - my_z)
        diag = (1 - my_x, my_y, 1 - my_z)
        r_idx = 2 * my_x + my_z
        x_nbr_r = 2 * (1 - my_x) + my_z
        z_nbr_r = 2 * my_x + (1 - my_z)
        diag_r = 2 * (1 - my_x) + (1 - my_z)

        my_start = my_y * S_HALF + r_idx * Q
        peer_start = (1 - my_y) * S_HALF + r_idx * Q

        o_copies = []
        for b in range(B):
            for i, start in enumerate((peer_start, my_start)):
                cp = pltpu.make_async_copy(
                    o_ref.at[b, pl.ds(start, Q)],
                    o_vmem.at[b, i],
                    o_copy_sems.at[b, i],
                )
                cp.start()
                o_copies.append(cp)

        barrier_sem = pltpu.get_barrier_semaphore()
        for nbr in (y_peer, x_nbr, z_nbr, diag):
            pl.semaphore_signal(
                barrier_sem, inc=1, device_id=nbr,
                device_id_type=pl.DeviceIdType.MESH,
            )
        pl.semaphore_wait(barrier_sem, 4)
        for cp in o_copies:
            cp.wait()

        def y_rdma(s):
            return pltpu.make_async_remote_copy(
                src_ref=psend.at[s], dst_ref=precv.at[s],
                send_sem=y_send_sems.at[s], recv_sem=y_recv_sems.at[s],
                device_id=y_peer, device_id_type=pl.DeviceIdType.MESH,
            )

        for s in range(NSUB):
            for b in range(B):
                psend[s, b, :, :] = jnp.dot(
                    o_vmem[b, 0, pl.ds(s * QS, QS)],
                    wo_ref[...],
                    preferred_element_type=jnp.float32,
                ).astype(jnp.bfloat16)
            y_rdma(s).start()

        ag_sends = []
        for s in range(NSUB):
            row0 = r_idx * Q + s * QS
            mine = [
                jnp.dot(
                    o_vmem[b, 1, pl.ds(s * QS, QS)],
                    wo_ref[...],
                    preferred_element_type=jnp.float32,
                )
                for b in range(B)
            ]
            y_rdma(s).wait_recv()
            for b in range(B):
                q = mine[b] + precv[s, b].astype(jnp.float32)
                out_ref[b, pl.ds(row0, QS), :] = q
                agsend[s, b, :, :] = q.astype(jnp.bfloat16)
            for i, nbr in enumerate((x_nbr, z_nbr, diag)):
                snd = pltpu.make_async_remote_copy(
                    src_ref=agsend.at[s], dst_ref=agrecv.at[r_idx, s],
                    send_sem=ag_send_sems.at[i, s],
                    recv_sem=ag_recv_sems.at[r_idx, s],
                    device_id=nbr, device_id_type=pl.DeviceIdType.MESH,
                )
                snd.start()
                ag_sends.append(snd)

        for src_r in (x_nbr_r, z_nbr_r, diag_r):
            for s in range(NSUB):
                recv = pltpu.make_async_remote_copy(
                    src_ref=agsend.at[s], dst_ref=agrecv.at[src_r, s],
                    send_sem=ag_send_sems.at[0, s],
                    recv_sem=ag_recv_sems.at[src_r, s],
                    device_id=y_peer, device_id_type=pl.DeviceIdType.MESH,
                )
                recv.wait_recv()
                for b in range(B):
                    out_ref[b, pl.ds(src_r * Q + s * QS, QS), :] = (
                        agrecv[src_r, s, b, :, :].astype(jnp.float32)
                    )

        for s in range(NSUB):
            y_rdma(s).wait_send()
        for snd in ag_sends:
            snd.wait_send()

    return pl.pallas_call(
        body,
        out_shape=jax.ShapeDtypeStruct((B, S_HALF, F), jnp.float32),
        in_specs=[
            pl.BlockSpec(memory_space=pl.ANY),
            pl.BlockSpec(memory_space=pltpu.VMEM),
        ],
        out_specs=pl.BlockSpec(memory_space=pltpu.VMEM),
        scratch_shapes=[
            pltpu.VMEM((B, 2, Q, HD_LOCAL), jnp.float32),
            pltpu.VMEM((NSUB, B, QS, F), jnp.bfloat16),
            pltpu.VMEM((NSUB, B, QS, F), jnp.bfloat16),
            pltpu.VMEM((NSUB, B, QS, F), jnp.bfloat16),
            pltpu.VMEM((4, NSUB, B, QS, F), jnp.bfloat16),
            pltpu.SemaphoreType.DMA((B, 2)),
            pltpu.SemaphoreType.DMA((NSUB,)),
            pltpu.SemaphoreType.DMA((NSUB,)),
            pltpu.SemaphoreType.DMA((3, NSUB)),
            pltpu.SemaphoreType.DMA((4, NSUB)),
        ],
        compiler_params=pltpu.CompilerParams(collective_id=0),
    )(O_flat, Wo)


# device time: 43650 ns/iter; 1.2513x vs baseline; 1.2513x over previous
import jax
import jax.numpy as jnp
from jax import lax
from jax.experimental import pallas as pl
from jax.experimental.pallas import tpu as pltpu

B, S, H, D, F = 2, 1024, 16, 64, 2048
S_HALF = S // 2
Q = S_HALF // 4
NSUB = 4
QS = Q // NSUB


def kernel(O, Wo):
    my_y = lax.axis_index("y")
    r_idx = 2 * lax.axis_index("x") + lax.axis_index("z")
    my_start = my_y * S_HALF + r_idx * Q
    peer_start = (1 - my_y) * S_HALF + r_idx * Q
    o_peer = lax.dynamic_slice_in_dim(O, peer_start, Q, axis=1).reshape(
        B, Q, H * D
    )
    o_mine = lax.dynamic_slice_in_dim(O, my_start, Q, axis=1).reshape(
        B, Q, H * D
    )

    def body(o_peer_ref, o_mine_ref, wo_ref, out_ref, psend, precv, agsend,
             agrecv, y_send_sems, y_recv_sems, ag_send_sems, ag_recv_sems):
        o_blks = (o_peer_ref, o_mine_ref)
        my_x = lax.axis_index("x")
        my_y = lax.axis_index("y")
        my_z = lax.axis_index("z")
        y_peer = (my_x, 1 - my_y, my_z)
        x_nbr = (1 - my_x, my_y, my_z)
        z_nbr = (my_x, my_y, 1 - my_z)
        diag = (1 - my_x, my_y, 1 - my_z)
        r_idx = 2 * my_x + my_z
        x_nbr_r = 2 * (1 - my_x) + my_z
        z_nbr_r = 2 * my_x + (1 - my_z)
        diag_r = 2 * (1 - my_x) + (1 - my_z)

        barrier_sem = pltpu.get_barrier_semaphore()
        for nbr in (y_peer, x_nbr, z_nbr, diag):
            pl.semaphore_signal(
                barrier_sem, inc=1, device_id=nbr,
                device_id_type=pl.DeviceIdType.MESH,
            )
        pl.semaphore_wait(barrier_sem, 4)

        def partial_dot(b, blk, s):
            return jnp.dot(
                o_blks[blk][b, pl.ds(s * QS, QS), :],
                wo_ref[...],
                preferred_element_type=jnp.float32,
            )

        def y_rdma(s):
            return pltpu.make_async_remote_copy(
                src_ref=psend.at[s], dst_ref=precv.at[s],
                send_sem=y_send_sems.at[s], recv_sem=y_recv_sems.at[s],
                device_id=y_peer, device_id_type=pl.DeviceIdType.MESH,
            )

        for s in range(NSUB):
            for b in range(B):
                psend[s, b, :, :] = partial_dot(b, 0, s).astype(jnp.bfloat16)
            y_rdma(s).start()

        ag_sends = []
        for s in range(NSUB):
            row0 = r_idx * Q + s * QS
            mine = [partial_dot(b, 1, s) for b in range(B)]
            y_rdma(s).wait_recv()
            for b in range(B):
                q = mine[b] + precv[s, b].astype(jnp.float32)
                out_ref[b, pl.ds(row0, QS), :] = q
                agsend[s, b, :, :] = q.astype(jnp.bfloat16)
            for i, nbr in enumerate((x_nbr, z_nbr, diag)):
                snd = pltpu.make_async_remote_copy(
                    src_ref=agsend.at[s], dst_ref=agrecv.at[r_idx, s],
                    send_sem=ag_send_sems.at[i, s],
                    recv_sem=ag_recv_sems.at[r_idx, s],
                    device_id=nbr, device_id_type=pl.DeviceIdType.MESH,
                )
                snd.start()
                ag_sends.append(snd)

        for src_r in (x_nbr_r, z_nbr_r, diag_r):
            for s in range(NSUB):
                recv = pltpu.make_async_remote_copy(
                    src_ref=agsend.at[s], dst_ref=agrecv.at[src_r, s],
                    send_sem=ag_send_sems.at[0, s],
                    recv_sem=ag_recv_sems.at[src_r, s],
                    device_id=y_peer, device_id_type=pl.DeviceIdType.MESH,
                )
                recv.wait_recv()
                for b in range(B):
                    out_ref[b, pl.ds(src_r * Q + s * QS, QS), :] = (
                        agrecv[src_r, s, b, :, :].astype(jnp.float32)
                    )

        for s in range(NSUB):
            y_rdma(s).wait_send()
        for snd in ag_sends:
            snd.wait_send()

    return pl.pallas_call(
        body,
        out_shape=jax.ShapeDtypeStruct((B, S_HALF, F), jnp.float32),
        in_specs=[
            pl.BlockSpec(memory_space=pltpu.VMEM),
            pl.BlockSpec(memory_space=pltpu.VMEM),
            pl.BlockSpec(memory_space=pltpu.VMEM),
        ],
        out_specs=pl.BlockSpec(memory_space=pltpu.VMEM),
        scratch_shapes=[
            pltpu.VMEM((NSUB, B, QS, F), jnp.bfloat16),
            pltpu.VMEM((NSUB, B, QS, F), jnp.bfloat16),
            pltpu.VMEM((NSUB, B, QS, F), jnp.bfloat16),
            pltpu.VMEM((4, NSUB, B, QS, F), jnp.bfloat16),
            pltpu.SemaphoreType.DMA((NSUB,)),
            pltpu.SemaphoreType.DMA((NSUB,)),
            pltpu.SemaphoreType.DMA((3, NSUB)),
            pltpu.SemaphoreType.DMA((4, NSUB)),
        ],
        compiler_params=pltpu.CompilerParams(collective_id=0),
    )(o_peer, o_mine, Wo)
